# Initial kernel scaffold; baseline (speedup 1.0000x reference)
import jax
import jax.numpy as jnp
from jax.experimental import pallas as pl


def kernel(user, item, by_src, by_time, pby_src, pby_time, W_user, W_recipe, W_user_update, W_recipe_update, W_agg_user, W_agg_recipe, user_date_emb, user_date_emb_k, recipe_date_emb, recipe_date_emb_k):
    raise NotImplementedError("write your pallas kernel here")


# speedup vs baseline: 3.5131x; 3.5131x over previous
import functools
import math

import jax
import jax.numpy as jnp
from jax import lax
from jax.experimental import pallas as pl
from jax.experimental.pallas import tpu as pltpu
from jax.experimental.pallas import tpu_sc as plsc

_HIGHEST = lax.Precision.HIGHEST

_SC_CORES = 2
_SC_SUBCORES = 16
_SC_WORKERS = _SC_CORES * _SC_SUBCORES
_GATHER_CHUNK = 128


def _sc_gather_rows(table, idx_flat):
  e_total, h = table.shape[0], table.shape[1]
  e = idx_flat.shape[0]
  assert e % _GATHER_CHUNK == 0
  n_chunks = e // _GATHER_CHUNK
  n_iters = (n_chunks + _SC_WORKERS - 1) // _SC_WORKERS
  del e_total

  mesh = plsc.VectorSubcoreMesh(core_axis_name="c", subcore_axis_name="s")

  @functools.partial(
      pl.kernel,
      mesh=mesh,
      out_type=jax.ShapeDtypeStruct((e, h), jnp.float32),
      scratch_types=[
          pltpu.VMEM((_GATHER_CHUNK,), jnp.int32),
          pltpu.VMEM((_GATHER_CHUNK, h), jnp.float32),
          pltpu.SemaphoreType.DMA,
      ],
  )
  def gather_kernel(table_hbm, idx_hbm, out_hbm, idx_v, rows_v, sem):
    wid = lax.axis_index("s") * _SC_CORES + lax.axis_index("c")

    def body(i, carry):
      cid = wid + i * _SC_WORKERS

      @pl.when(cid < n_chunks)
      def _():
        base = cid * _GATHER_CHUNK
        pltpu.sync_copy(idx_hbm.at[pl.ds(base, _GATHER_CHUNK)], idx_v)
        pltpu.async_copy(table_hbm.at[idx_v], rows_v, sem).wait()
        pltpu.sync_copy(rows_v, out_hbm.at[pl.ds(base, _GATHER_CHUNK)])

      return carry

    lax.fori_loop(0, n_iters, body, 0)

  return gather_kernel(table, idx_flat)


def _pick_block(n, cap=256):
  for b in range(cap, 7, -8):
    if n % b == 0:
      return b
  return n


def _project(x, w):
  n, k = x.shape
  k2, m = w.shape
  b = _pick_block(n, cap=1024)

  def body(x_ref, w_ref, o_ref):
    o_ref[...] = jnp.dot(x_ref[...], w_ref[...], precision=_HIGHEST,
                         preferred_element_type=jnp.float32)

  return pl.pallas_call(
      body,
      grid=(n // b,),
      in_specs=[
          pl.BlockSpec((b, k), lambda i: (i, 0)),
          pl.BlockSpec((k2, m), lambda i: (0, 0)),
      ],
      out_specs=pl.BlockSpec((b, m), lambda i: (i, 0)),
      out_shape=jax.ShapeDtypeStruct((n, m), jnp.float32),
      compiler_params=pltpu.CompilerParams(
          dimension_semantics=("parallel",)),
  )(x, w)


def _reduce_update(mb, dst_h, time, emb_d, embk_d, w_agg, dst_raw, w_upd):
  n, d, h = mb.shape
  b = _pick_block(n, cap=256)
  inv_sqrt_h = 1.0 / math.sqrt(h)

  def body(mb_ref, dsth_ref, time_ref, embd_ref, embkd_ref, wagg_ref,
           draw_ref, wupd_ref, out_ref):
    t = time_ref[...]
    mbv = mb_ref[...]
    dsth = dsth_ref[...]

    ti = t[:, :, None]
    tj = t[:, None, :]
    ii = lax.broadcasted_iota(jnp.int32, (b, d, d), 1)
    jj = lax.broadcasted_iota(jnp.int32, (b, d, d), 2)
    less = jnp.where((tj < ti) | ((tj == ti) & (jj < ii)),
                     jnp.int32(1), jnp.int32(0))
    rank = jnp.sum(less, axis=2)
    reorder = jnp.int32(d - 1) - rank

    kk = lax.broadcasted_iota(jnp.int32, (b, d, d), 2)
    oh = jnp.where(reorder[:, :, None] == kk, jnp.float32(1.0),
                   jnp.float32(0.0))

    p = lax.dot_general(dsth, embd_ref[...], (((1,), (1,)), ((), ())),
                        precision=_HIGHEST,
                        preferred_element_type=jnp.float32)
    p_g = jnp.sum(oh * p[:, None, :], axis=2)
    dot_mb = jnp.sum(mbv * dsth[:, None, :], axis=2)
    e_ui = (p_g + dot_mb) * inv_sqrt_h
    m0 = jnp.max(e_ui, axis=1, keepdims=True)
    ex0 = jnp.exp(e_ui - m0)
    alpha = ex0 / jnp.sum(ex0, axis=1, keepdims=True)

    beta = jnp.sum(oh * alpha[:, :, None], axis=1)
    h_long = (jnp.sum(alpha[:, :, None] * mbv, axis=1) +
              jnp.dot(beta, embkd_ref[...], precision=_HIGHEST,
                      preferred_element_type=jnp.float32))

    tmax = jnp.max(t, axis=1, keepdims=True)
    de = lax.broadcasted_iota(jnp.int32, (b, d), 1)
    first = jnp.min(jnp.where(t == tmax, de, jnp.int32(d)),
                    axis=1, keepdims=True)
    last_oh = jnp.where(de == first, jnp.float32(1.0), jnp.float32(0.0))
    last_emb = jnp.sum(last_oh[:, :, None] * mbv, axis=1)
    e1 = jnp.sum(mbv * last_emb[:, None, :], axis=2) * inv_sqrt_h
    m1 = jnp.max(e1, axis=1, keepdims=True)
    ex1 = jnp.exp(e1 - m1)
    alpha1 = ex1 / jnp.sum(ex1, axis=1, keepdims=True)
    h_short = jnp.sum(alpha1[:, :, None] * mbv, axis=1)

    agg = (jnp.dot(h_long, wagg_ref[0:h, :], precision=_HIGHEST,
                   preferred_element_type=jnp.float32) +
           jnp.dot(h_short, wagg_ref[h:2 * h, :], precision=_HIGHEST,
                   preferred_element_type=jnp.float32))
    out = jnp.tanh(
        jnp.dot(agg, wupd_ref[0:h, :], precision=_HIGHEST,
                preferred_element_type=jnp.float32) +
        jnp.dot(draw_ref[...], wupd_ref[h:2 * h, :], precision=_HIGHEST,
                preferred_element_type=jnp.float32))
    out_ref[...] = out

  return pl.pallas_call(
      body,
      grid=(n // b,),
      in_specs=[
          pl.BlockSpec((b, d, h), lambda i: (i, 0, 0)),
          pl.BlockSpec((b, h), lambda i: (i, 0)),
          pl.BlockSpec((b, d), lambda i: (i, 0)),
          pl.BlockSpec((d, h), lambda i: (0, 0)),
          pl.BlockSpec((d, h), lambda i: (0, 0)),
          pl.BlockSpec((2 * h, h), lambda i: (0, 0)),
          pl.BlockSpec((b, h), lambda i: (i, 0)),
          pl.BlockSpec((2 * h, h), lambda i: (0, 0)),
      ],
      out_specs=pl.BlockSpec((b, h), lambda i: (i, 0)),
      out_shape=jax.ShapeDtypeStruct((n, h), jnp.float32),
      compiler_params=pltpu.CompilerParams(
          dimension_semantics=("parallel",)),
  )(mb, dst_h, time, emb_d, embk_d, w_agg, dst_raw, w_upd)


def kernel(user, item, by_src, by_time, pby_src, pby_time, W_user, W_recipe,
           W_user_update, W_recipe_update, W_agg_user, W_agg_recipe,
           user_date_emb, user_date_emb_k, recipe_date_emb,
           recipe_date_emb_k):
  nu, h = user.shape
  ni = item.shape[0]
  d = by_src.shape[1]

  user_h = _project(user, W_user)
  item_h = _project(item, W_recipe)

  mb_user = _sc_gather_rows(item_h, by_src.reshape(-1)).reshape(nu, d, h)
  mb_item = _sc_gather_rows(user_h, pby_src.reshape(-1)).reshape(ni, d, h)

  user_out = _reduce_update(mb_user, user_h, by_time, user_date_emb[:d],
                            user_date_emb_k[:d], W_agg_user, user,
                            W_user_update)
  item_out = _reduce_update(mb_item, item_h, pby_time, recipe_date_emb[:d],
                            recipe_date_emb_k[:d], W_agg_recipe, item,
                            W_recipe_update)
  return (user_out, item_out)
